# baseline (speedup 1.0000x reference)
import functools

import jax
import jax.numpy as jnp
from jax import lax
from jax.experimental import pallas as pl
from jax.experimental.pallas import tpu as pltpu
from jax.experimental.pallas import tpu_sc as plsc

B = 16384
F = 26
K = 16
NC = 2
NS = 16
NW = NC * NS
BW = B // NW
CHUNK = 128
CHW = BW * F // CHUNK
G = 64
NG = BW // G
NCHG = G * F // CHUNK

VOC = 1000000
TBLK = 128
YROWS = VOC * K // CHUNK
YPB = TBLK * K // CHUNK


def _tc_tr_body(x_ref, y_ref):
    x = x_ref[...]
    parts = [jnp.take_along_axis(
                 x, jnp.broadcast_to(jnp.arange(jj, TBLK, 8), (16, TBLK // 8)),
                 axis=1, mode="promise_in_bounds").T
             for jj in range(8)]
    y_ref[...] = jnp.concatenate(parts, axis=1)


def _fm_body(idx_hbm, w_hbm, v_hbm, out_hbm, idx_v,
             vbuf0, wbuf0, vbuf1, wbuf1, obuf, sem0, sem1):
    wid = lax.axis_index("s") * NC + lax.axis_index("c")
    pltpu.sync_copy(idx_hbm.at[wid], idx_v)

    def dmas(g, vbuf, wbuf, sem):
        cbase = g * NCHG
        out = []
        for j in range(NCHG):
            ixrow = idx_v.at[cbase + j]
            out.append(pltpu.make_async_copy(
                v_hbm.at[ixrow], vbuf.at[pl.ds(j * CHUNK, CHUNK)], sem))
            out.append(pltpu.make_async_copy(
                w_hbm.at[ixrow], wbuf.at[pl.ds(j * CHUNK, CHUNK)], sem))
        return out

    def fire(g, vbuf, wbuf, sem):
        for c in dmas(g, vbuf, wbuf, sem):
            c.start()

    def drain(g, vbuf, wbuf, sem):
        for c in dmas(g, vbuf, wbuf, sem):
            c.wait()

    def compute(g, vbuf, wbuf):
        lanes = lax.iota(jnp.int32, 16)

        def tile(t, c2):
            def sample(s16, rvec):
                base = (t * 16 + s16) * F
                v0 = vbuf[base, :]
                sv = v0
                ssv = v0 * v0
                for f in range(1, F):
                    v = vbuf[base + f, :]
                    sv = sv + v
                    ssv = ssv + v * v
                wa = wbuf[pl.ds(base, 16)]
                wb = wbuf[pl.ds(base + 16, 16)]
                wbm = jnp.where(lanes < F - 16, wb, 0.0)
                total = 0.5 * (sv * sv - ssv) + wa + wbm
                for sh in (8, 4, 2, 1):
                    perm = (lanes + sh) & 15
                    total = total + total.at[perm].get(
                        mode="promise_in_bounds")
                return jnp.where(lanes == s16, total, rvec)

            rvec = lax.fori_loop(0, 16, sample,
                                 jnp.zeros((16,), jnp.float32))
            obuf[pl.ds(g * G + t * 16, 16)] = rvec
            return c2

        lax.fori_loop(0, G // 16, tile, 0)

    fire(0, vbuf0, wbuf0, sem0)

    def pair(p, carry):
        g0 = 2 * p
        fire(g0 + 1, vbuf1, wbuf1, sem1)
        drain(g0, vbuf0, wbuf0, sem0)
        compute(g0, vbuf0, wbuf0)

        @pl.when(g0 + 2 < NG)
        def _():
            fire(g0 + 2, vbuf0, wbuf0, sem0)

        drain(g0 + 1, vbuf1, wbuf1, sem1)
        compute(g0 + 1, vbuf1, wbuf1)
        return carry

    lax.fori_loop(0, NG // 2, pair, 0)
    pltpu.sync_copy(obuf, out_hbm.at[pl.ds(wid * BW, BW)])


def _mesh():
    return plsc.VectorSubcoreMesh(core_axis_name="c", subcore_axis_name="s",
                                  num_cores=NC, num_subcores=NS)


@functools.partial(jax.jit, static_argnames=())
def _fm_call(idx, w_flat, vt):
    nblk = (VOC + TBLK - 1) // TBLK
    transpose = pl.pallas_call(
        _tc_tr_body,
        grid=(nblk,),
        in_specs=[pl.BlockSpec((16, TBLK), lambda i: (0, i))],
        out_specs=pl.BlockSpec((YPB, CHUNK), lambda i: (i, 0)),
        out_shape=jax.ShapeDtypeStruct((YROWS, CHUNK), jnp.float32),
    )
    y = transpose(vt)
    table = y.reshape(VOC, K)

    run = pl.kernel(
        _fm_body,
        out_type=jax.ShapeDtypeStruct((B,), jnp.float32),
        mesh=_mesh(),
        scratch_types=[
            pltpu.VMEM((CHW, CHUNK), jnp.int32),
            pltpu.VMEM((G * F, K), jnp.float32),
            pltpu.VMEM((G * F + 16,), jnp.float32),
            pltpu.VMEM((G * F, K), jnp.float32),
            pltpu.VMEM((G * F + 16,), jnp.float32),
            pltpu.VMEM((BW,), jnp.float32),
            pltpu.SemaphoreType.DMA,
            pltpu.SemaphoreType.DMA,
        ],
        compiler_params=pltpu.CompilerParams(use_tc_tiling_on_sc=False),
    )
    return run(idx, w_flat, table)


def kernel(inputs, w0, w, V):
    idx = inputs.astype(jnp.int32).reshape(NW, CHW, CHUNK)
    w_flat = w.reshape(w.shape[0])
    vt = V.T
    out = _fm_call(idx, w_flat, vt)
    return out[:, None] + w0


# speedup vs baseline: 31.3930x; 31.3930x over previous
import functools

import jax
import jax.numpy as jnp
from jax import lax
from jax.experimental import pallas as pl
from jax.experimental.pallas import tpu as pltpu
from jax.experimental.pallas import tpu_sc as plsc

B = 16384
F = 26
K = 16
NC = 2
NS = 16
NW = NC * NS
BW = B // NW
CHUNK = 128
CHW = BW * F // CHUNK
G = 64
NG = BW // G
NCHG = G * F // CHUNK

VOC = 1000000
WBLK = 1024
NBLK_MAIN = 960
NBLK_FULL = VOC // WBLK
T512_BASE = NBLK_FULL * WBLK
TAIL_BASE = (VOC // CHUNK) * CHUNK
YROWS = VOC * K // CHUNK
YPB = WBLK * K // CHUNK


def _tr_body(vt_hbm, tail_hbm, y_hbm, xb0, xb1, yb0, yb1,
             sem0, sem1, osem0, osem1):
    wid = lax.axis_index("s") * NC + lax.axis_index("c")
    lanes = lax.iota(jnp.int32, 16)

    def in_dmas(block, xb, sem, width=WBLK):
        col0 = block * WBLK
        out = []
        for kb in range(2):
            out.append(pltpu.make_async_copy(
                vt_hbm.at[pl.ds(kb * 8, 8), pl.ds(col0, width)],
                xb.at[pl.ds(kb * 8, 8), pl.ds(0, width)], sem))
        return out

    def out_dma(block, yb, osem, width=WBLK):
        rows = width * K // CHUNK
        return pltpu.make_async_copy(
            yb.at[pl.ds(0, rows)], y_hbm.at[pl.ds(block * YPB, rows)], osem)

    def compute(xb, yb, width=WBLK):
        def grp_body(g, c):
            base = g * 16
            cur = [xb[k, pl.ds(base, 16)] for k in range(16)]
            for s in (1, 2, 4, 8):
                perm = lanes ^ s
                sel = (lanes & s) == 0
                new = list(cur)
                for j in range(16):
                    if (j & s) == 0:
                        pj = cur[j ^ s].at[perm].get(
                            mode="promise_in_bounds")
                        pk = cur[j].at[perm].get(mode="promise_in_bounds")
                        new[j] = jnp.where(sel, cur[j], pj)
                        new[j ^ s] = jnp.where(sel, pk, cur[j ^ s])
                cur = new
            row = g * 2
            for j in range(16):
                yb[row + j // 8, pl.ds((j % 8) * K, K)] = cur[j]
            return c

        lax.fori_loop(0, width // 16, grp_body, 0)

    def fire_in(block, xb, sem):
        for c in in_dmas(block, xb, sem):
            c.start()

    def drain_in(block, xb, sem):
        for c in in_dmas(block, xb, sem):
            c.wait()

    fire_in(wid, xb0, sem0)
    nslots = NBLK_MAIN // NW

    def pair(p, carry):
        b0 = 2 * p * NW + wid
        fire_in(b0 + NW, xb1, sem1)
        drain_in(b0, xb0, sem0)

        @pl.when(p > 0)
        def _():
            out_dma(b0 - 2 * NW, yb0, osem0).wait()

        compute(xb0, yb0)
        out_dma(b0, yb0, osem0).start()

        @pl.when(2 * p + 2 < nslots)
        def _():
            fire_in(b0 + 2 * NW, xb0, sem0)

        drain_in(b0 + NW, xb1, sem1)

        @pl.when(p > 0)
        def _():
            out_dma(b0 - NW, yb1, osem1).wait()

        compute(xb1, yb1)
        out_dma(b0 + NW, yb1, osem1).start()
        return carry

    lax.fori_loop(0, nslots // 2, pair, 0)
    out_dma((nslots - 2) * NW + wid, yb0, osem0).wait()
    out_dma((nslots - 1) * NW + wid, yb1, osem1).wait()

    @pl.when(wid < 16)
    def _():
        block = NBLK_MAIN + wid
        for c in in_dmas(block, xb0, sem0):
            c.start()
        for c in in_dmas(block, xb0, sem0):
            c.wait()
        compute(xb0, yb0)
        out_dma(block, yb0, osem0).start()
        out_dma(block, yb0, osem0).wait()

    @pl.when(wid == 16)
    def _():
        for c in in_dmas(NBLK_FULL, xb0, sem0, width=512):
            c.start()
        for c in in_dmas(NBLK_FULL, xb0, sem0, width=512):
            c.wait()
        compute(xb0, yb0, width=512)
        out_dma(NBLK_FULL, yb0, osem0, width=512).start()
        out_dma(NBLK_FULL, yb0, osem0, width=512).wait()

    @pl.when(wid == 17)
    def _():
        pltpu.make_async_copy(
            tail_hbm.at[pl.ds(0, 8)],
            y_hbm.at[pl.ds(TAIL_BASE * K // CHUNK, 8)], sem0).start()
        pltpu.make_async_copy(
            tail_hbm.at[pl.ds(0, 8)],
            y_hbm.at[pl.ds(TAIL_BASE * K // CHUNK, 8)], sem0).wait()


def _fm_body(idx_hbm, w_hbm, v_hbm, out_hbm, idx_v,
             vbuf0, wbuf0, vbuf1, wbuf1, obuf, sem0, sem1):
    wid = lax.axis_index("s") * NC + lax.axis_index("c")
    pltpu.sync_copy(idx_hbm.at[wid], idx_v)

    def dmas(g, vbuf, wbuf, sem):
        cbase = g * NCHG
        out = []
        for j in range(NCHG):
            ixrow = idx_v.at[cbase + j]
            out.append(pltpu.make_async_copy(
                v_hbm.at[ixrow], vbuf.at[pl.ds(j * CHUNK, CHUNK)], sem))
            out.append(pltpu.make_async_copy(
                w_hbm.at[ixrow], wbuf.at[pl.ds(j * CHUNK, CHUNK)], sem))
        return out

    def fire(g, vbuf, wbuf, sem):
        for c in dmas(g, vbuf, wbuf, sem):
            c.start()

    def drain(g, vbuf, wbuf, sem):
        for c in dmas(g, vbuf, wbuf, sem):
            c.wait()

    def compute(g, vbuf, wbuf):
        lanes = lax.iota(jnp.int32, 16)

        def tile(t, c2):
            def sample(s16, rvec):
                base = (t * 16 + s16) * F
                v0 = vbuf[base, :]
                sv = v0
                ssv = v0 * v0
                for f in range(1, F):
                    v = vbuf[base + f, :]
                    sv = sv + v
                    ssv = ssv + v * v
                wa = wbuf[pl.ds(base, 16)]
                wb = wbuf[pl.ds(base + 16, 16)]
                wbm = jnp.where(lanes < F - 16, wb, 0.0)
                total = 0.5 * (sv * sv - ssv) + wa + wbm
                for sh in (8, 4, 2, 1):
                    perm = (lanes + sh) & 15
                    total = total + total.at[perm].get(
                        mode="promise_in_bounds")
                return jnp.where(lanes == s16, total, rvec)

            rvec = lax.fori_loop(0, 16, sample,
                                 jnp.zeros((16,), jnp.float32))
            obuf[pl.ds(g * G + t * 16, 16)] = rvec
            return c2

        lax.fori_loop(0, G // 16, tile, 0)

    fire(0, vbuf0, wbuf0, sem0)

    def pair(p, carry):
        g0 = 2 * p
        fire(g0 + 1, vbuf1, wbuf1, sem1)
        drain(g0, vbuf0, wbuf0, sem0)
        compute(g0, vbuf0, wbuf0)

        @pl.when(g0 + 2 < NG)
        def _():
            fire(g0 + 2, vbuf0, wbuf0, sem0)

        drain(g0 + 1, vbuf1, wbuf1, sem1)
        compute(g0 + 1, vbuf1, wbuf1)
        return carry

    lax.fori_loop(0, NG // 2, pair, 0)
    pltpu.sync_copy(obuf, out_hbm.at[pl.ds(wid * BW, BW)])


def _mesh():
    return plsc.VectorSubcoreMesh(core_axis_name="c", subcore_axis_name="s",
                                  num_cores=NC, num_subcores=NS)


@functools.partial(jax.jit, static_argnames=())
def _fm_call(idx, w_flat, vt, tail):
    transpose = pl.kernel(
        _tr_body,
        out_type=jax.ShapeDtypeStruct((YROWS, CHUNK), jnp.float32),
        mesh=_mesh(),
        scratch_types=[
            pltpu.VMEM((16, WBLK + 8), jnp.float32),
            pltpu.VMEM((16, WBLK + 8), jnp.float32),
            pltpu.VMEM((YPB, CHUNK), jnp.float32),
            pltpu.VMEM((YPB, CHUNK), jnp.float32),
            pltpu.SemaphoreType.DMA,
            pltpu.SemaphoreType.DMA,
            pltpu.SemaphoreType.DMA,
            pltpu.SemaphoreType.DMA,
        ],
        compiler_params=pltpu.CompilerParams(use_tc_tiling_on_sc=True,
                                             needs_layout_passes=False),
    )
    y = transpose(vt, tail)
    table = y.reshape(VOC, K)

    run = pl.kernel(
        _fm_body,
        out_type=jax.ShapeDtypeStruct((B,), jnp.float32),
        mesh=_mesh(),
        scratch_types=[
            pltpu.VMEM((CHW, CHUNK), jnp.int32),
            pltpu.VMEM((G * F, K), jnp.float32),
            pltpu.VMEM((G * F + 16,), jnp.float32),
            pltpu.VMEM((G * F, K), jnp.float32),
            pltpu.VMEM((G * F + 16,), jnp.float32),
            pltpu.VMEM((BW,), jnp.float32),
            pltpu.SemaphoreType.DMA,
            pltpu.SemaphoreType.DMA,
        ],
        compiler_params=pltpu.CompilerParams(use_tc_tiling_on_sc=False),
    )
    return run(idx, w_flat, table)


def kernel(inputs, w0, w, V):
    idx = inputs.astype(jnp.int32).reshape(NW, CHW, CHUNK)
    w_flat = w.reshape(w.shape[0])
    vt = V.T
    tail = jnp.pad(V[TAIL_BASE:, :],
                   ((0, CHUNK - (VOC - TAIL_BASE)), (0, 0))).reshape(16, 128)
    out = _fm_call(idx, w_flat, vt, tail)
    return out[:, None] + w0
